# baseline (speedup 1.0000x reference)
import functools

import jax
import jax.numpy as jnp
from jax import lax
from jax.experimental import pallas as pl
from jax.experimental.pallas import tpu as pltpu
from jax.experimental.pallas import tpu_sc as plsc

B, S, DM, H = 1, 2048, 768, 12
D = DM // H
G = 32
N1 = S // G
TOP1 = 8
WIN = 512
TQ = 256
NT = S // TQ
TA = 512
NA = S // TA
BPT = TA // G
NEG = -1e30


def _mm(a, b, dims):
    return lax.dot_general(a.astype(jnp.bfloat16), b.astype(jnp.bfloat16),
                           dims, preferred_element_type=jnp.float32)


def _qkv_body(x_ref, wq_ref, wk_ref, wv_ref, cos_ref, sin_ref,
              q_ref, k_ref, v_ref):
    xt = x_ref[...]
    cos = cos_ref[...]
    sin = sin_ref[...]
    qf = _mm(xt, wq_ref[...], (((1,), (1,)), ((), ())))
    kf = _mm(xt, wk_ref[...], (((1,), (1,)), ((), ())))
    vf = _mm(xt, wv_ref[...], (((1,), (1,)), ((), ())))
    for h in range(H):
        qh = qf[:, h * D:(h + 1) * D]
        kh = kf[:, h * D:(h + 1) * D]
        qrot = jnp.concatenate([-qh[:, D // 2:], qh[:, :D // 2]], axis=1)
        krot = jnp.concatenate([-kh[:, D // 2:], kh[:, :D // 2]], axis=1)
        q_ref[h] = qh * cos + qrot * sin
        k_ref[h] = kh * cos + krot * sin
        v_ref[h] = vf[:, h * D:(h + 1) * D]


def _route_body(q_ref, kr_ref, wk1_ref, sc_ref):
    k1 = _mm(kr_ref[...], wk1_ref[...], (((1,), (1,)), ((), ())))
    sc_ref[...] = lax.dot_general(k1, q_ref[...], (((1,), (1,)), ((), ())),
                                  preferred_element_type=jnp.float32,
                                  precision=lax.Precision.HIGHEST)


QW = 128
NCHUNK = H * S // QW
CPW = NCHUNK // 32


def _sc_select(scores2d):
    mesh = plsc.VectorSubcoreMesh(core_axis_name="c", subcore_axis_name="s")
    kern = functools.partial(
        pl.kernel,
        mesh=mesh,
        out_type=jax.ShapeDtypeStruct((H * 2 * S,), jnp.int32),
        scratch_types=[
            pltpu.VMEM((N1, QW), jnp.float32),
            pltpu.VMEM((QW,), jnp.int32),
            pltpu.VMEM((QW,), jnp.int32),
        ],
    )(_sc_select_body)
    return kern(scores2d)


def _sc_select_body(scores_hbm, out_hbm, buf, olo, ohi):
    nc = 2
    wid = lax.axis_index("s") * nc + lax.axis_index("c")

    def chunk(cc, carry):
        c = wid * CPW + cc
        h = c // (S // QW)
        col0 = (c % (S // QW)) * QW
        pltpu.sync_copy(
            scores_hbm.at[pl.ds(h * N1, N1), pl.ds(col0, QW)], buf)
        for g in range(QW // 16):
            def ld(b):
                return buf[b, g * 16:(g + 1) * 16]
            t = [jnp.full((16,), NEG, jnp.float32) for _ in range(TOP1)]
            for b in range(N1):
                v = ld(b)
                for i in range(TOP1 - 1, 0, -1):
                    t[i] = jnp.maximum(t[i], jnp.minimum(t[i - 1], v))
                t[0] = jnp.maximum(t[0], v)
            thr = t[TOP1 - 1]
            ngt = jnp.zeros((16,), jnp.int32)
            for b in range(N1):
                ngt = ngt + jnp.where(ld(b) > thr, 1, 0)
            quota = TOP1 - ngt
            ceq = jnp.zeros((16,), jnp.int32)
            lo = jnp.zeros((16,), jnp.int32)
            hi = jnp.zeros((16,), jnp.int32)
            for b in range(N1):
                v = ld(b)
                eq = v == thr
                hit = (v > thr) | (eq & (ceq < quota))
                ceq = ceq + jnp.where(eq, 1, 0)
                word = (1 << (b % 32)) & 0xFFFFFFFF
                if word >= 2 ** 31:
                    word -= 2 ** 32
                add = jnp.where(hit, jnp.int32(word), jnp.int32(0))
                if b < G:
                    lo = lo | add
                else:
                    hi = hi | add
            olo[g * 16:(g + 1) * 16] = lo
            ohi[g * 16:(g + 1) * 16] = hi
        pltpu.sync_copy(olo, out_hbm.at[pl.ds(2 * h * S + col0, QW)])
        pltpu.sync_copy(ohi, out_hbm.at[pl.ds((2 * h + 1) * S + col0, QW)])
        return carry

    lax.fori_loop(0, CPW, chunk, 0)


def _attn_body(q_ref, k_ref, v_ref, sel_ref, o_ref):
    i = pl.program_id(1)
    scale = D ** -0.5
    qt = q_ref[...] * scale
    bits = sel_ref[...]
    sh = lax.broadcasted_iota(jnp.int32, (G, TA), 0)
    lo = jnp.broadcast_to(bits[0:1, :], (G, TA))
    hi = jnp.broadcast_to(bits[1:2, :], (G, TA))
    selt = jnp.concatenate([(lo >> sh) & 1, (hi >> sh) & 1],
                           axis=0).astype(jnp.float32)
    ri = lax.broadcasted_iota(jnp.int32, (TA, TA), 0)
    ci = lax.broadcasted_iota(jnp.int32, (TA, TA), 1)
    causal_pen = jnp.where(ci <= ri, 0.0, NEG).astype(jnp.float32)
    anti_pen = jnp.where(ci > ri, 0.0, NEG).astype(jnp.float32)
    eb = lax.broadcasted_iota(jnp.int32, (N1, TA), 0)
    ec = lax.broadcasted_iota(jnp.int32, (N1, TA), 1) // G

    def tile(j, carry, mode):
        m, l, acc = carry
        kt = k_ref[pl.ds(j * TA, TA), :]
        vt = v_ref[pl.ds(j * TA, TA), :]
        s = _mm(qt, kt, (((1,), (1,)), ((), ())))
        if mode != "diag":
            ej = (eb == j * BPT + ec).astype(jnp.float32)
            sel_exp = _mm(selt, ej, (((0,), (0,)), ((), ())))
            sel_pen = (sel_exp - 1.0) * -NEG
            if mode == "win":
                s = s + jnp.maximum(anti_pen, sel_pen)
            else:
                s = s + sel_pen
        else:
            s = s + causal_pen
        m2 = jnp.maximum(m, jnp.max(s, axis=1, keepdims=True))
        alpha = jnp.exp(m - m2)
        p = jnp.exp(s - m2)
        l2 = l * alpha + jnp.sum(p, axis=1, keepdims=True)
        acc2 = acc * alpha + _mm(p, vt, (((1,), (0,)), ((), ())))
        return m2, l2, acc2

    m0 = jnp.full((TA, 1), NEG, dtype=jnp.float32)
    l0 = jnp.zeros((TA, 1), dtype=jnp.float32)
    a0 = jnp.zeros((TA, D), dtype=jnp.float32)
    carry = (m0, l0, a0)
    carry = lax.fori_loop(0, jnp.maximum(i - 1, 0),
                          lambda j, c: tile(j, c, "far"), carry)
    carry = lax.cond(i >= 1,
                     lambda c: tile(i - 1, c, "win"),
                     lambda c: c, carry)
    m, l, acc = tile(i, carry, "diag")
    o_ref[...] = acc / l


def _proj_body(o_ref, wo_ref, y_ref):
    acc = jnp.zeros((TQ, DM), dtype=jnp.float32)
    for h in range(H):
        acc = acc + _mm(o_ref[h], wo_ref[:, h * D:(h + 1) * D],
                        (((1,), (1,)), ((), ())))
    y_ref[...] = acc


def _pipeline(x2, wq, wk, wv, wo, wk1, interpret=False):
    inv_freq = 1.0 / (10000.0 ** (jnp.arange(0, D, 2, dtype=jnp.float32) / D))
    t = jnp.arange(S, dtype=jnp.float32)
    freqs = jnp.outer(t, inv_freq)
    emb = jnp.concatenate([freqs, freqs], axis=-1)
    cos = jnp.cos(emb)
    sin = jnp.sin(emb)

    q, k, v = pl.pallas_call(
        _qkv_body,
        grid=(NT,),
        in_specs=[
            pl.BlockSpec((TQ, DM), lambda i: (i, 0)),
            pl.BlockSpec((DM, DM), lambda i: (0, 0)),
            pl.BlockSpec((DM, DM), lambda i: (0, 0)),
            pl.BlockSpec((DM, DM), lambda i: (0, 0)),
            pl.BlockSpec((TQ, D), lambda i: (i, 0)),
            pl.BlockSpec((TQ, D), lambda i: (i, 0)),
        ],
        out_specs=[
            pl.BlockSpec((H, TQ, D), lambda i: (0, i, 0)),
            pl.BlockSpec((H, TQ, D), lambda i: (0, i, 0)),
            pl.BlockSpec((H, TQ, D), lambda i: (0, i, 0)),
        ],
        out_shape=[jax.ShapeDtypeStruct((H, S, D), jnp.float32)] * 3,
        interpret=interpret,
    )(x2, wq, wk, wv, cos, sin)

    kr = k.reshape(H, N1, G * D)

    scores = pl.pallas_call(
        _route_body,
        grid=(H,),
        in_specs=[
            pl.BlockSpec((None, S, D), lambda h: (h, 0, 0)),
            pl.BlockSpec((None, N1, G * D), lambda h: (h, 0, 0)),
            pl.BlockSpec((D, G * D), lambda h: (0, 0)),
        ],
        out_specs=pl.BlockSpec((None, N1, S), lambda h: (h, 0, 0)),
        out_shape=jax.ShapeDtypeStruct((H, N1, S), jnp.float32),
        interpret=interpret,
    )(q, kr, wk1)

    sel = _sc_select(scores.reshape(H * N1, S)).reshape(H, 2, S)

    o = pl.pallas_call(
        _attn_body,
        grid=(H, NA),
        in_specs=[
            pl.BlockSpec((None, TA, D), lambda h, i: (h, i, 0)),
            pl.BlockSpec((None, S, D), lambda h, i: (h, 0, 0)),
            pl.BlockSpec((None, S, D), lambda h, i: (h, 0, 0)),
            pl.BlockSpec((None, 2, TA), lambda h, i: (h, 0, i)),
        ],
        out_specs=pl.BlockSpec((None, TA, D), lambda h, i: (h, i, 0)),
        out_shape=jax.ShapeDtypeStruct((H, S, D), jnp.float32),
        interpret=interpret,
    )(q, k, v, sel)

    y = pl.pallas_call(
        _proj_body,
        grid=(NT,),
        in_specs=[
            pl.BlockSpec((H, TQ, D), lambda i: (0, i, 0)),
            pl.BlockSpec((DM, DM), lambda i: (0, 0)),
        ],
        out_specs=pl.BlockSpec((TQ, DM), lambda i: (i, 0)),
        out_shape=jax.ShapeDtypeStruct((S, DM), jnp.float32),
        interpret=interpret,
    )(o, wo)
    return y


@jax.jit
def kernel(x, Wq, Wk, Wv, Wo, Wk1, Wv1, Wk2, Wv2):
    del Wv1, Wk2, Wv2
    y = _pipeline(x[0], Wq, Wk, Wv, Wo, Wk1)
    return y[None]


# speedup vs baseline: 1.0079x; 1.0079x over previous
import functools

import jax
import jax.numpy as jnp
from jax import lax
from jax.experimental import pallas as pl
from jax.experimental.pallas import tpu as pltpu
from jax.experimental.pallas import tpu_sc as plsc

B, S, DM, H = 1, 2048, 768, 12
D = DM // H
G = 32
N1 = S // G
TOP1 = 8
WIN = 512
TQ = 256
NT = S // TQ
TA = 512
NA = S // TA
BPT = TA // G
NEG = -1e30


def _mm(a, b, dims):
    return lax.dot_general(a.astype(jnp.bfloat16), b.astype(jnp.bfloat16),
                           dims, preferred_element_type=jnp.float32)


def _qkv_body(x_ref, wq_ref, wk_ref, wv_ref, cos_ref, sin_ref,
              q_ref, k_ref, v_ref):
    xt = x_ref[...]
    cos = cos_ref[...]
    sin = sin_ref[...]
    qf = _mm(xt, wq_ref[...], (((1,), (1,)), ((), ())))
    kf = _mm(xt, wk_ref[...], (((1,), (1,)), ((), ())))
    vf = _mm(xt, wv_ref[...], (((1,), (1,)), ((), ())))
    for h in range(H):
        qh = qf[:, h * D:(h + 1) * D]
        kh = kf[:, h * D:(h + 1) * D]
        qrot = jnp.concatenate([-qh[:, D // 2:], qh[:, :D // 2]], axis=1)
        krot = jnp.concatenate([-kh[:, D // 2:], kh[:, :D // 2]], axis=1)
        q_ref[h] = qh * cos + qrot * sin
        k_ref[h] = kh * cos + krot * sin
        v_ref[h] = vf[:, h * D:(h + 1) * D]


def _route_body(q_ref, kr_ref, wk1_ref, sc_ref):
    k1 = _mm(kr_ref[...], wk1_ref[...], (((1,), (1,)), ((), ())))
    sc_ref[...] = lax.dot_general(k1, q_ref[...], (((1,), (1,)), ((), ())),
                                  preferred_element_type=jnp.float32,
                                  precision=lax.Precision.HIGHEST)


QW = 128
NCHUNK = H * S // QW
CPW = NCHUNK // 32


def _sc_select(scores2d):
    mesh = plsc.VectorSubcoreMesh(core_axis_name="c", subcore_axis_name="s")
    kern = functools.partial(
        pl.kernel,
        mesh=mesh,
        out_type=jax.ShapeDtypeStruct((H * 2 * S,), jnp.int32),
        scratch_types=[
            pltpu.VMEM((N1, QW), jnp.float32),
            pltpu.VMEM((QW,), jnp.int32),
            pltpu.VMEM((QW,), jnp.int32),
        ],
    )(_sc_select_body)
    return kern(scores2d)


def _sc_select_body(scores_hbm, out_hbm, buf, olo, ohi):
    nc = 2
    wid = lax.axis_index("s") * nc + lax.axis_index("c")

    def chunk(cc, carry):
        c = wid * CPW + cc
        h = c // (S // QW)
        col0 = (c % (S // QW)) * QW
        pltpu.sync_copy(
            scores_hbm.at[pl.ds(h * N1, N1), pl.ds(col0, QW)], buf)
        for g in range(QW // 16):
            def ld(b):
                return buf[b, g * 16:(g + 1) * 16]
            t = [jnp.full((16,), NEG, jnp.float32) for _ in range(TOP1)]
            for b in range(N1):
                v = ld(b)
                for i in range(TOP1 - 1, 0, -1):
                    t[i] = jnp.maximum(t[i], jnp.minimum(t[i - 1], v))
                t[0] = jnp.maximum(t[0], v)
            thr = t[TOP1 - 1]
            ngt = jnp.zeros((16,), jnp.int32)
            for b in range(N1):
                ngt = ngt + jnp.where(ld(b) > thr, 1, 0)
            quota = TOP1 - ngt
            ceq = jnp.zeros((16,), jnp.int32)
            lo = jnp.zeros((16,), jnp.int32)
            hi = jnp.zeros((16,), jnp.int32)
            for b in range(N1):
                v = ld(b)
                eq = v == thr
                hit = (v > thr) | (eq & (ceq < quota))
                ceq = ceq + jnp.where(eq, 1, 0)
                word = (1 << (b % 32)) & 0xFFFFFFFF
                if word >= 2 ** 31:
                    word -= 2 ** 32
                add = jnp.where(hit, jnp.int32(word), jnp.int32(0))
                if b < G:
                    lo = lo | add
                else:
                    hi = hi | add
            olo[g * 16:(g + 1) * 16] = lo
            ohi[g * 16:(g + 1) * 16] = hi
        pltpu.sync_copy(olo, out_hbm.at[pl.ds(2 * h * S + col0, QW)])
        pltpu.sync_copy(ohi, out_hbm.at[pl.ds((2 * h + 1) * S + col0, QW)])
        return carry

    lax.fori_loop(0, CPW, chunk, 0)


def _flash_tile(qt, k_ref, v_ref, j, carry, pen):
    m, l, acc = carry
    kt = k_ref[pl.ds(j * TA, TA), :]
    vt = v_ref[pl.ds(j * TA, TA), :]
    s = _mm(qt, kt, (((1,), (1,)), ((), ()))) + pen
    m2 = jnp.maximum(m, jnp.max(s, axis=1, keepdims=True))
    alpha = jnp.exp(m - m2)
    p = jnp.exp(s - m2)
    l2 = l * alpha + jnp.sum(p, axis=1, keepdims=True)
    acc2 = acc * alpha + _mm(p, vt, (((1,), (0,)), ((), ())))
    return m2, l2, acc2


def _attn_near_body(q_ref, k_ref, v_ref, on_ref, mn_ref, ln_ref):
    i = pl.program_id(1)
    qt = q_ref[...] * (D ** -0.5)
    ri = lax.broadcasted_iota(jnp.int32, (TA, TA), 0)
    ci = lax.broadcasted_iota(jnp.int32, (TA, TA), 1)
    causal_pen = jnp.where(ci <= ri, 0.0, NEG).astype(jnp.float32)
    anti_pen = jnp.where(ci > ri, 0.0, NEG).astype(jnp.float32)

    m0 = jnp.full((TA, 1), NEG, dtype=jnp.float32)
    l0 = jnp.zeros((TA, 1), dtype=jnp.float32)
    a0 = jnp.zeros((TA, D), dtype=jnp.float32)
    carry = lax.cond(
        i >= 1,
        lambda c: _flash_tile(qt, k_ref, v_ref, i - 1, c, anti_pen),
        lambda c: c, (m0, l0, a0))
    m, l, acc = _flash_tile(qt, k_ref, v_ref, i, carry, causal_pen)
    on_ref[...] = acc
    mn_ref[...] = jnp.broadcast_to(m, (TA, 8))
    ln_ref[...] = jnp.broadcast_to(l, (TA, 8))


def _attn_far_body(q_ref, k_ref, v_ref, sel_ref, on_ref, mn_ref, ln_ref,
                   o_ref):
    i = pl.program_id(1)
    qt = q_ref[...] * (D ** -0.5)
    bits = sel_ref[...]
    sh = lax.broadcasted_iota(jnp.int32, (G, TA), 0)
    lo = jnp.broadcast_to(bits[0:1, :], (G, TA))
    hi = jnp.broadcast_to(bits[1:2, :], (G, TA))
    selt = jnp.concatenate([(lo >> sh) & 1, (hi >> sh) & 1],
                           axis=0).astype(jnp.float32)
    ri = lax.broadcasted_iota(jnp.int32, (TA, TA), 0)
    ci = lax.broadcasted_iota(jnp.int32, (TA, TA), 1)
    causal_pen = jnp.where(ci <= ri, 0.0, NEG).astype(jnp.float32)
    eb = lax.broadcasted_iota(jnp.int32, (N1, TA), 0)
    ec = lax.broadcasted_iota(jnp.int32, (N1, TA), 1) // G

    def sel_pen(j):
        ej = (eb == j * BPT + ec).astype(jnp.float32)
        sel_exp = _mm(selt, ej, (((0,), (0,)), ((), ())))
        return (sel_exp - 1.0) * -NEG

    m0 = jnp.full((TA, 1), NEG, dtype=jnp.float32)
    l0 = jnp.zeros((TA, 1), dtype=jnp.float32)
    a0 = jnp.zeros((TA, D), dtype=jnp.float32)
    carry = lax.fori_loop(
        0, jnp.maximum(i - 1, 0),
        lambda j, c: _flash_tile(qt, k_ref, v_ref, j, c, sel_pen(j)),
        (m0, l0, a0))
    m, l, acc = lax.cond(
        i >= 1,
        lambda c: _flash_tile(qt, k_ref, v_ref, i - 1, c,
                              sel_pen(i - 1) + causal_pen),
        lambda c: c, carry)

    mn = mn_ref[...][:, 0:1]
    ln = ln_ref[...][:, 0:1]
    mt = jnp.maximum(mn, m)
    an = jnp.exp(mn - mt)
    af = jnp.exp(m - mt)
    o_ref[...] = ((on_ref[...] * an + acc * af) /
                  (ln * an + l * af))


def _proj_body(o_ref, wo_ref, y_ref):
    acc = jnp.zeros((TQ, DM), dtype=jnp.float32)
    for h in range(H):
        acc = acc + _mm(o_ref[h], wo_ref[:, h * D:(h + 1) * D],
                        (((1,), (1,)), ((), ())))
    y_ref[...] = acc


def _pipeline(x2, wq, wk, wv, wo, wk1, interpret=False):
    inv_freq = 1.0 / (10000.0 ** (jnp.arange(0, D, 2, dtype=jnp.float32) / D))
    t = jnp.arange(S, dtype=jnp.float32)
    freqs = jnp.outer(t, inv_freq)
    emb = jnp.concatenate([freqs, freqs], axis=-1)
    cos = jnp.cos(emb)
    sin = jnp.sin(emb)

    q, k, v = pl.pallas_call(
        _qkv_body,
        grid=(NT,),
        in_specs=[
            pl.BlockSpec((TQ, DM), lambda i: (i, 0)),
            pl.BlockSpec((DM, DM), lambda i: (0, 0)),
            pl.BlockSpec((DM, DM), lambda i: (0, 0)),
            pl.BlockSpec((DM, DM), lambda i: (0, 0)),
            pl.BlockSpec((TQ, D), lambda i: (i, 0)),
            pl.BlockSpec((TQ, D), lambda i: (i, 0)),
        ],
        out_specs=[
            pl.BlockSpec((H, TQ, D), lambda i: (0, i, 0)),
            pl.BlockSpec((H, TQ, D), lambda i: (0, i, 0)),
            pl.BlockSpec((H, TQ, D), lambda i: (0, i, 0)),
        ],
        out_shape=[jax.ShapeDtypeStruct((H, S, D), jnp.float32)] * 3,
        interpret=interpret,
    )(x2, wq, wk, wv, cos, sin)

    kr = k.reshape(H, N1, G * D)

    scores = pl.pallas_call(
        _route_body,
        grid=(H,),
        in_specs=[
            pl.BlockSpec((None, S, D), lambda h: (h, 0, 0)),
            pl.BlockSpec((None, N1, G * D), lambda h: (h, 0, 0)),
            pl.BlockSpec((D, G * D), lambda h: (0, 0)),
        ],
        out_specs=pl.BlockSpec((None, N1, S), lambda h: (h, 0, 0)),
        out_shape=jax.ShapeDtypeStruct((H, N1, S), jnp.float32),
        interpret=interpret,
    )(q, kr, wk1)

    sel = _sc_select(scores.reshape(H * N1, S)).reshape(H, 2, S)

    on, mn, ln = pl.pallas_call(
        _attn_near_body,
        grid=(H, NA),
        in_specs=[
            pl.BlockSpec((None, TA, D), lambda h, i: (h, i, 0)),
            pl.BlockSpec((None, S, D), lambda h, i: (h, 0, 0)),
            pl.BlockSpec((None, S, D), lambda h, i: (h, 0, 0)),
        ],
        out_specs=[
            pl.BlockSpec((None, TA, D), lambda h, i: (h, i, 0)),
            pl.BlockSpec((None, TA, 8), lambda h, i: (h, i, 0)),
            pl.BlockSpec((None, TA, 8), lambda h, i: (h, i, 0)),
        ],
        out_shape=[
            jax.ShapeDtypeStruct((H, S, D), jnp.float32),
            jax.ShapeDtypeStruct((H, S, 8), jnp.float32),
            jax.ShapeDtypeStruct((H, S, 8), jnp.float32),
        ],
        interpret=interpret,
    )(q, k, v)

    o = pl.pallas_call(
        _attn_far_body,
        grid=(H, NA),
        in_specs=[
            pl.BlockSpec((None, TA, D), lambda h, i: (h, i, 0)),
            pl.BlockSpec((None, S, D), lambda h, i: (h, 0, 0)),
            pl.BlockSpec((None, S, D), lambda h, i: (h, 0, 0)),
            pl.BlockSpec((None, 2, TA), lambda h, i: (h, 0, i)),
            pl.BlockSpec((None, TA, D), lambda h, i: (h, i, 0)),
            pl.BlockSpec((None, TA, 8), lambda h, i: (h, i, 0)),
            pl.BlockSpec((None, TA, 8), lambda h, i: (h, i, 0)),
        ],
        out_specs=pl.BlockSpec((None, TA, D), lambda h, i: (h, i, 0)),
        out_shape=jax.ShapeDtypeStruct((H, S, D), jnp.float32),
        interpret=interpret,
    )(q, k, v, sel, on, mn, ln)

    y = pl.pallas_call(
        _proj_body,
        grid=(NT,),
        in_specs=[
            pl.BlockSpec((H, TQ, D), lambda i: (0, i, 0)),
            pl.BlockSpec((DM, DM), lambda i: (0, 0)),
        ],
        out_specs=pl.BlockSpec((TQ, DM), lambda i: (i, 0)),
        out_shape=jax.ShapeDtypeStruct((S, DM), jnp.float32),
        interpret=interpret,
    )(o, wo)
    return y


@jax.jit
def kernel(x, Wq, Wk, Wv, Wo, Wk1, Wv1, Wk2, Wv2):
    del Wv1, Wk2, Wv2
    y = _pipeline(x[0], Wq, Wk, Wv, Wo, Wk1)
    return y[None]
